# Initial kernel scaffold; baseline (speedup 1.0000x reference)
import jax
import jax.numpy as jnp
from jax.experimental import pallas as pl


def kernel(roi_features, union_features, rel_pair_idxs, obj_pred_labels, W_obj, W_rel, W_s, W_o, W_r, W_obj_cls, W_rel_cls, freq_table):
    raise NotImplementedError("write your pallas kernel here")


# speedup vs baseline: 4.2207x; 4.2207x over previous
import functools
import math

import jax
import jax.numpy as jnp
from jax import lax
from jax.experimental import pallas as pl
from jax.experimental.pallas import tpu as pltpu
from jax.experimental.pallas import tpu_sc as plsc

F32 = jnp.float32
I32 = jnp.int32

_NW = 32
_TILES = 16
_C = 80


def _sc_mesh():
    return plsc.VectorSubcoreMesh(core_axis_name="c", subcore_axis_name="s")


def _sc_gather_sum(S, O, src, dst):
    E = src.shape[0]
    D = S.shape[1]
    per_w = E // _NW
    nch = per_w // _C

    @functools.partial(
        pl.kernel,
        mesh=_sc_mesh(),
        compiler_params=pltpu.CompilerParams(needs_layout_passes=False),
        out_type=jax.ShapeDtypeStruct((E, D), F32),
        scratch_types=[
            pltpu.VMEM((_C,), I32),
            pltpu.VMEM((_C,), I32),
            pltpu.VMEM((_C, D), F32),
            pltpu.VMEM((_C, D), F32),
        ],
    )
    def k(S_hbm, O_hbm, src_hbm, dst_hbm, out_hbm, sidx, didx, ra, rb):
        cid = lax.axis_index("c")
        sid = lax.axis_index("s")
        wid = sid * 2 + cid
        base = wid * per_w

        def chunk(c, carry):
            off = pl.multiple_of(base + c * _C, 8)
            pltpu.sync_copy(src_hbm.at[pl.ds(off, _C)], sidx)
            pltpu.sync_copy(dst_hbm.at[pl.ds(off, _C)], didx)
            pltpu.sync_copy(S_hbm.at[sidx], ra)
            pltpu.sync_copy(O_hbm.at[didx], rb)

            def addrow(r, c2):
                for g in range(D // 16):
                    sl = pl.ds(g * 16, 16)
                    ra[r, sl] = ra[r, sl] + rb[r, sl]
                return c2

            lax.fori_loop(0, _C, addrow, 0)
            pltpu.sync_copy(ra, out_hbm.at[pl.ds(off, _C)])
            return carry

        lax.fori_loop(0, nch, chunk, 0)

    return k(S, O, src, dst)


def _sc_scatter_both(rmsg, src, dst, n_pad):
    E, D = rmsg.shape
    per_w = E // _NW
    nch = per_w // _C
    rows_per_tile = n_pad // _TILES
    stage_rows = rows_per_tile // 5

    @functools.partial(
        pl.kernel,
        mesh=_sc_mesh(),
        compiler_params=pltpu.CompilerParams(needs_layout_passes=False),
        out_type=jax.ShapeDtypeStruct((2 * n_pad, D), F32),
        scratch_types=[
            pltpu.VMEM((_C,), I32),
            pltpu.VMEM((_C,), I32),
            pltpu.VMEM((_C, D), F32),
            pltpu.VMEM((stage_rows, D), F32),
            pltpu.VMEM_SHARED((n_pad, D), F32),
        ],
    )
    def k(rmsg_hbm, src_hbm, dst_hbm, out_hbm, sidx, didx, rows, stage, agg):
        cid = lax.axis_index("c")
        sid = lax.axis_index("s")
        wid = sid * 2 + cid
        base = wid * per_w

        def zrow(r, carry):
            for g in range(D // 16):
                stage[r, pl.ds(g * 16, 16)] = jnp.zeros((16,), F32)
            return carry

        lax.fori_loop(0, stage_rows, zrow, 0)
        for kk in range(5):
            r0 = sid * rows_per_tile + kk * stage_rows
            pltpu.sync_copy(stage, agg.at[pl.ds(r0, stage_rows)])
        plsc.subcore_barrier()

        def chunk(c, carry):
            off = pl.multiple_of(base + c * _C, 8)
            pltpu.sync_copy(src_hbm.at[pl.ds(off, _C)], sidx)
            pltpu.sync_copy(dst_hbm.at[pl.ds(off, _C)], didx)
            pltpu.sync_copy(rmsg_hbm.at[pl.ds(off, _C)], rows)
            pltpu.sync_copy(rows, agg.at[sidx], add=True)
            pltpu.sync_copy(rows, agg.at[didx], add=True)
            return carry

        lax.fori_loop(0, nch, chunk, 0)
        plsc.subcore_barrier()

        for kk in range(5):
            r0 = sid * rows_per_tile + kk * stage_rows
            pltpu.sync_copy(agg.at[pl.ds(r0, stage_rows)], stage)
            pltpu.sync_copy(stage, out_hbm.at[pl.ds(cid * n_pad + r0, stage_rows)])

    return k(rmsg, src, dst)


def _sc_pair_deg(src, dst, labels, freq_pad, n_cls, n_pad):
    E = src.shape[0]
    n_obj = labels.shape[0]
    Fp = freq_pad.shape[1]
    per_w = E // _NW
    nch = per_w // _C
    rows_per_tile = n_pad // _TILES
    stage_rows = rows_per_tile // 5

    @functools.partial(
        pl.kernel,
        mesh=_sc_mesh(),
        compiler_params=pltpu.CompilerParams(needs_layout_passes=False,
                                             use_tc_tiling_on_sc=False),
        out_type=[
            jax.ShapeDtypeStruct((E, Fp), F32),
            jax.ShapeDtypeStruct((2 * n_pad, 16), F32),
        ],
        scratch_types=[
            pltpu.VMEM((_C,), I32),
            pltpu.VMEM((_C,), I32),
            pltpu.VMEM((_C,), I32),
            pltpu.VMEM((n_obj,), I32),
            pltpu.VMEM((_C, Fp), F32),
            pltpu.VMEM((_C, 16), F32),
            pltpu.VMEM((stage_rows, 16), F32),
            pltpu.VMEM_SHARED((n_pad, 16), F32),
        ],
    )
    def k(src_hbm, dst_hbm, lab_hbm, freq_hbm, bias_hbm, deg_hbm,
          sidx, didx, pidx, labv, frows, ones, stage, deg):
        cid = lax.axis_index("c")
        sid = lax.axis_index("s")
        wid = sid * 2 + cid
        base = wid * per_w

        pltpu.sync_copy(lab_hbm, labv)

        def orow(r, carry):
            ones[r, :] = jnp.ones((16,), F32)
            return carry

        lax.fori_loop(0, _C, orow, 0)

        def zrow(r, carry):
            stage[r, :] = jnp.zeros((16,), F32)
            return carry

        lax.fori_loop(0, stage_rows, zrow, 0)
        for kk in range(5):
            r0 = sid * rows_per_tile + kk * stage_rows
            pltpu.sync_copy(stage, deg.at[pl.ds(r0, stage_rows)])
        plsc.subcore_barrier()

        def chunk(c, carry):
            off = pl.multiple_of(base + c * _C, 8)
            pltpu.sync_copy(src_hbm.at[pl.ds(off, _C)], sidx)
            pltpu.sync_copy(dst_hbm.at[pl.ds(off, _C)], didx)
            pltpu.sync_copy(ones, deg.at[sidx], add=True)
            pltpu.sync_copy(ones, deg.at[didx], add=True)
            for g in range(_C // 16):
                sl = pl.ds(g * 16, 16)
                ls = plsc.load_gather(labv, [sidx[sl]])
                ld = plsc.load_gather(labv, [didx[sl]])
                pidx[sl] = ls * n_cls + ld
            pltpu.sync_copy(freq_hbm.at[pidx], frows)
            pltpu.sync_copy(frows, bias_hbm.at[pl.ds(off, _C)])
            return carry

        lax.fori_loop(0, nch, chunk, 0)
        plsc.subcore_barrier()

        for kk in range(5):
            r0 = sid * rows_per_tile + kk * stage_rows
            pltpu.sync_copy(deg.at[pl.ds(r0, stage_rows)], stage)
            pltpu.sync_copy(stage, deg_hbm.at[pl.ds(cid * n_pad + r0, stage_rows)])

    return k(src, dst, labels, freq_pad)


_OBJ_BLK = 1280
_REL_BLK = 2560


def _tc_obj0(roi, W_obj, W_s, W_o):
    n, d = roi.shape
    dh = W_obj.shape[1]
    grid = n // _OBJ_BLK

    def body(x_ref, wobj_ref, ws_ref, wo_ref, h_ref, s_ref, o_ref):
        h = jnp.maximum(
            jnp.dot(x_ref[...], wobj_ref[...], preferred_element_type=F32), 0.0)
        h_ref[...] = h
        s_ref[...] = jnp.dot(h, ws_ref[...], preferred_element_type=F32)
        o_ref[...] = jnp.dot(h, wo_ref[...], preferred_element_type=F32)

    w_spec = pl.BlockSpec((d, dh), lambda i: (0, 0))
    row_spec = pl.BlockSpec((_OBJ_BLK, dh), lambda i: (i, 0))
    return pl.pallas_call(
        body,
        grid=(grid,),
        in_specs=[pl.BlockSpec((_OBJ_BLK, d), lambda i: (i, 0)),
                  w_spec, w_spec, w_spec],
        out_specs=[row_spec, row_spec, row_spec],
        out_shape=[jax.ShapeDtypeStruct((n, dh), F32)] * 3,
    )(roi, W_obj, W_s, W_o)


def _tc_rel_first(union, msg, W_rel, W_r):
    e, d = union.shape
    dh = W_rel.shape[1]
    grid = e // _REL_BLK

    def body(u_ref, m_ref, wrel_ref, wr_ref, h_ref, r_ref):
        h0 = jnp.maximum(
            jnp.dot(u_ref[...], wrel_ref[...], preferred_element_type=F32), 0.0)
        h = jnp.maximum(h0 + m_ref[...], 0.0)
        h_ref[...] = h
        r_ref[...] = jnp.dot(h, wr_ref[...], preferred_element_type=F32)

    blk = pl.BlockSpec((_REL_BLK, dh), lambda i: (i, 0))
    w_spec = pl.BlockSpec((d, dh), lambda i: (0, 0))
    return pl.pallas_call(
        body,
        grid=(grid,),
        in_specs=[pl.BlockSpec((_REL_BLK, d), lambda i: (i, 0)), blk,
                  w_spec, pl.BlockSpec((dh, dh), lambda i: (0, 0))],
        out_specs=[blk, blk],
        out_shape=[jax.ShapeDtypeStruct((e, dh), F32)] * 2,
    )(union, msg, W_rel, W_r)


def _tc_rel_next(rel_h, msg, W_r):
    e, dh = rel_h.shape
    grid = e // _REL_BLK

    def body(rh_ref, m_ref, wr_ref, r_ref):
        h = jnp.maximum(rh_ref[...] + m_ref[...], 0.0)
        r_ref[...] = jnp.dot(h, wr_ref[...], preferred_element_type=F32)

    blk = pl.BlockSpec((_REL_BLK, dh), lambda i: (i, 0))
    return pl.pallas_call(
        body,
        grid=(grid,),
        in_specs=[blk, blk, pl.BlockSpec((dh, dh), lambda i: (0, 0))],
        out_specs=blk,
        out_shape=jax.ShapeDtypeStruct((e, dh), F32),
    )(rel_h, msg, W_r)


def _tc_obj_update(obj_h, aggp, degp, W_s, W_o):
    n, dh = obj_h.shape
    grid = n // _OBJ_BLK

    def body(h_ref, a_ref, d_ref, ws_ref, wo_ref, h2_ref, s_ref, o_ref):
        a = a_ref[0] + a_ref[1]
        deg = jnp.maximum(d_ref[0, :, 0:1] + d_ref[1, :, 0:1], 1.0)
        h = jnp.maximum(h_ref[...] + a / deg, 0.0)
        h2_ref[...] = h
        s_ref[...] = jnp.dot(h, ws_ref[...], preferred_element_type=F32)
        o_ref[...] = jnp.dot(h, wo_ref[...], preferred_element_type=F32)

    blk = pl.BlockSpec((_OBJ_BLK, dh), lambda i: (i, 0))
    w_spec = pl.BlockSpec((dh, dh), lambda i: (0, 0))
    return pl.pallas_call(
        body,
        grid=(grid,),
        in_specs=[blk,
                  pl.BlockSpec((2, _OBJ_BLK, dh), lambda i: (0, i, 0)),
                  pl.BlockSpec((2, _OBJ_BLK, 16), lambda i: (0, i, 0)),
                  w_spec, w_spec],
        out_specs=[blk, blk, blk],
        out_shape=[jax.ShapeDtypeStruct((n, dh), F32)] * 3,
    )(obj_h, aggp, degp, W_s, W_o)


def _tc_obj_final(obj_h, aggp, degp, W_cls_pad):
    n, dh = obj_h.shape
    ncp = W_cls_pad.shape[1]
    grid = n // _OBJ_BLK

    def body(h_ref, a_ref, d_ref, wc_ref, out_ref):
        a = a_ref[0] + a_ref[1]
        deg = jnp.maximum(d_ref[0, :, 0:1] + d_ref[1, :, 0:1], 1.0)
        h = jnp.maximum(h_ref[...] + a / deg, 0.0)
        out_ref[...] = jnp.dot(h, wc_ref[...], preferred_element_type=F32)

    blk = pl.BlockSpec((_OBJ_BLK, dh), lambda i: (i, 0))
    return pl.pallas_call(
        body,
        grid=(grid,),
        in_specs=[blk,
                  pl.BlockSpec((2, _OBJ_BLK, dh), lambda i: (0, i, 0)),
                  pl.BlockSpec((2, _OBJ_BLK, 16), lambda i: (0, i, 0)),
                  pl.BlockSpec((dh, ncp), lambda i: (0, 0))],
        out_specs=pl.BlockSpec((_OBJ_BLK, ncp), lambda i: (i, 0)),
        out_shape=jax.ShapeDtypeStruct((n, ncp), F32),
    )(obj_h, aggp, degp, W_cls_pad)


def _tc_rel_cls(rel_h, msg, bias, W_cls_pad):
    e, dh = rel_h.shape
    ncp = W_cls_pad.shape[1]
    grid = e // _REL_BLK

    def body(rh_ref, m_ref, b_ref, wc_ref, out_ref):
        h = jnp.maximum(rh_ref[...] + m_ref[...], 0.0)
        out_ref[...] = jnp.dot(h, wc_ref[...], preferred_element_type=F32) + b_ref[...]

    blk = pl.BlockSpec((_REL_BLK, dh), lambda i: (i, 0))
    bblk = pl.BlockSpec((_REL_BLK, ncp), lambda i: (i, 0))
    return pl.pallas_call(
        body,
        grid=(grid,),
        in_specs=[blk, blk, bblk, pl.BlockSpec((dh, ncp), lambda i: (0, 0))],
        out_specs=bblk,
        out_shape=jax.ShapeDtypeStruct((e, ncp), F32),
    )(rel_h, msg, bias, W_cls_pad)


def kernel(roi_features, union_features, rel_pair_idxs, obj_pred_labels,
           W_obj, W_rel, W_s, W_o, W_r, W_obj_cls, W_rel_cls, freq_table):
    n_obj, d_in = roi_features.shape
    E = union_features.shape[0]
    dh = W_obj.shape[1]
    n_obj_cls = W_obj_cls.shape[1]
    n_rel_cls = W_rel_cls.shape[1]
    n_cls = math.isqrt(freq_table.shape[0])

    n_pad = ((n_obj + _TILES * 40 - 1) // (_TILES * 40)) * (_TILES * 40)

    src = jnp.asarray(rel_pair_idxs[:, 0])
    dst = jnp.asarray(rel_pair_idxs[:, 1])
    roi_pad = jnp.pad(roi_features, ((0, n_pad - n_obj), (0, 0)))

    fp = 64
    freq_pad = jnp.pad(freq_table, ((0, 0), (0, fp - n_rel_cls)))
    ocp = ((n_obj_cls + 7) // 8) * 8
    W_obj_cls_pad = jnp.pad(W_obj_cls, ((0, 0), (0, ocp - n_obj_cls)))
    W_rel_cls_pad = jnp.pad(W_rel_cls, ((0, 0), (0, fp - n_rel_cls)))

    obj_h, S, O = _tc_obj0(roi_pad, W_obj, W_s, W_o)

    bias, degp_flat = _sc_pair_deg(src, dst, obj_pred_labels, freq_pad,
                                   n_cls, n_pad)
    degp = degp_flat.reshape(2, n_pad, 16)

    msg0 = _sc_gather_sum(S, O, src, dst)
    rel_h1, rmsg0 = _tc_rel_first(union_features, msg0, W_rel, W_r)
    aggp0 = _sc_scatter_both(rmsg0, src, dst, n_pad).reshape(2, n_pad, dh)
    obj_h1, S1, O1 = _tc_obj_update(obj_h, aggp0, degp, W_s, W_o)

    msg1 = _sc_gather_sum(S1, O1, src, dst)
    rmsg1 = _tc_rel_next(rel_h1, msg1, W_r)
    aggp1 = _sc_scatter_both(rmsg1, src, dst, n_pad).reshape(2, n_pad, dh)
    obj_logits = _tc_obj_final(obj_h1, aggp1, degp,
                               W_obj_cls_pad)[:n_obj, :n_obj_cls]

    rel_logits = _tc_rel_cls(rel_h1, msg1, bias, W_rel_cls_pad)[:, :n_rel_cls]

    return obj_logits, rel_logits


# speedup vs baseline: 5.7586x; 1.3644x over previous
import functools
import math

import jax
import jax.numpy as jnp
from jax import lax
from jax.experimental import pallas as pl
from jax.experimental.pallas import tpu as pltpu
from jax.experimental.pallas import tpu_sc as plsc

F32 = jnp.float32
I32 = jnp.int32

_NW = 32
_TILES = 16
_C = 80


def _sc_mesh():
    return plsc.VectorSubcoreMesh(core_axis_name="c", subcore_axis_name="s")


def _sc_gather_sum(S, O, src, dst):
    E = src.shape[0]
    D = S.shape[1]
    per_w = E // _NW
    nch = per_w // _C

    nb = 3

    @functools.partial(
        pl.kernel,
        mesh=_sc_mesh(),
        compiler_params=pltpu.CompilerParams(needs_layout_passes=False),
        out_type=jax.ShapeDtypeStruct((E, D), F32),
        scratch_types=(
            [pltpu.VMEM((_C,), I32) for _ in range(2 * nb)]
            + [pltpu.VMEM((_C, D), F32) for _ in range(2 * nb)]
            + [pltpu.SemaphoreType.DMA for _ in range(3 * nb)]
        ),
    )
    def k(S_hbm, O_hbm, src_hbm, dst_hbm, out_hbm, *scr):
        sidx = scr[0:nb]
        didx = scr[nb:2 * nb]
        ra = scr[2 * nb:3 * nb]
        rb = scr[3 * nb:4 * nb]
        si = scr[4 * nb:5 * nb]
        sg = scr[5 * nb:6 * nb]
        sw = scr[6 * nb:7 * nb]
        cid = lax.axis_index("c")
        sid = lax.axis_index("s")
        wid = sid * 2 + cid
        base = wid * per_w

        def off_of(kc):
            return pl.multiple_of(base + kc * _C, 8)

        def issue_idx(kc, b):
            pltpu.async_copy(src_hbm.at[pl.ds(off_of(kc), _C)], sidx[b], si[b])
            pltpu.async_copy(dst_hbm.at[pl.ds(off_of(kc), _C)], didx[b], si[b])

        def add_and_write(kc, b):
            def addrow(r, c2):
                for g in range(D // 16):
                    sl = pl.ds(g * 16, 16)
                    ra[b][r, sl] = ra[b][r, sl] + rb[b][r, sl]
                return c2

            lax.fori_loop(0, _C, addrow, 0)
            pltpu.async_copy(ra[b], out_hbm.at[pl.ds(off_of(kc), _C)], sw[b])

        issue_idx(0, 0)
        issue_idx(1, 1)

        n_outer = (nch + nb - 1) // nb

        def outer(j, carry):
            for b in range(nb):
                kc = j * nb + b
                bp = (b - 1) % nb

                @pl.when(kc < nch)
                def _():
                    pltpu.make_async_copy(
                        src_hbm.at[pl.ds(off_of(kc), _C)], sidx[b], si[b]).wait()
                    pltpu.make_async_copy(
                        dst_hbm.at[pl.ds(off_of(kc), _C)], didx[b], si[b]).wait()

                    @pl.when(kc >= nb)
                    def _():
                        pltpu.make_async_copy(
                            ra[b], out_hbm.at[pl.ds(off_of(kc), _C)],
                            sw[b]).wait()

                    pltpu.async_copy(S_hbm.at[sidx[b]], ra[b], sg[b])
                    pltpu.async_copy(O_hbm.at[didx[b]], rb[b], sg[b])

                    @pl.when(kc >= 1)
                    def _():
                        pltpu.make_async_copy(
                            S_hbm.at[sidx[bp]], ra[bp], sg[bp]).wait()
                        pltpu.make_async_copy(
                            O_hbm.at[didx[bp]], rb[bp], sg[bp]).wait()
                        add_and_write(kc - 1, bp)

                    @pl.when(kc + 2 < nch)
                    def _():
                        issue_idx(kc + 2, bp)
            return carry

        lax.fori_loop(0, n_outer, outer, 0)

        bl = (nch - 1) % nb
        pltpu.make_async_copy(S_hbm.at[sidx[bl]], ra[bl], sg[bl]).wait()
        pltpu.make_async_copy(O_hbm.at[didx[bl]], rb[bl], sg[bl]).wait()
        add_and_write(nch - 1, bl)
        for t in range(nb):
            bw = (nch - 1 - t) % nb
            pltpu.make_async_copy(
                ra[bw], out_hbm.at[pl.ds(off_of(nch - 1 - t), _C)],
                sw[bw]).wait()

    return k(S, O, src, dst)


def _sc_scatter_both(rmsg, src, dst, n_pad):
    E, D = rmsg.shape
    per_w = E // _NW
    nch = per_w // _C
    rows_per_tile = n_pad // _TILES
    stage_rows = rows_per_tile // 5

    @functools.partial(
        pl.kernel,
        mesh=_sc_mesh(),
        compiler_params=pltpu.CompilerParams(needs_layout_passes=False),
        out_type=jax.ShapeDtypeStruct((2 * n_pad, D), F32),
        scratch_types=[
            pltpu.VMEM((_C,), I32),
            pltpu.VMEM((_C,), I32),
            pltpu.VMEM((_C, D), F32),
            pltpu.VMEM((stage_rows, D), F32),
            pltpu.VMEM_SHARED((n_pad, D), F32),
        ],
    )
    def k(rmsg_hbm, src_hbm, dst_hbm, out_hbm, sidx, didx, rows, stage, agg):
        cid = lax.axis_index("c")
        sid = lax.axis_index("s")
        wid = sid * 2 + cid
        base = wid * per_w

        def zrow(r, carry):
            for g in range(D // 16):
                stage[r, pl.ds(g * 16, 16)] = jnp.zeros((16,), F32)
            return carry

        lax.fori_loop(0, stage_rows, zrow, 0)
        for kk in range(5):
            r0 = sid * rows_per_tile + kk * stage_rows
            pltpu.sync_copy(stage, agg.at[pl.ds(r0, stage_rows)])
        plsc.subcore_barrier()

        def chunk(c, carry):
            off = pl.multiple_of(base + c * _C, 8)
            pltpu.sync_copy(src_hbm.at[pl.ds(off, _C)], sidx)
            pltpu.sync_copy(dst_hbm.at[pl.ds(off, _C)], didx)
            pltpu.sync_copy(rmsg_hbm.at[pl.ds(off, _C)], rows)
            pltpu.sync_copy(rows, agg.at[sidx], add=True)
            pltpu.sync_copy(rows, agg.at[didx], add=True)
            return carry

        lax.fori_loop(0, nch, chunk, 0)
        plsc.subcore_barrier()

        for kk in range(5):
            r0 = sid * rows_per_tile + kk * stage_rows
            pltpu.sync_copy(agg.at[pl.ds(r0, stage_rows)], stage)
            pltpu.sync_copy(stage, out_hbm.at[pl.ds(cid * n_pad + r0, stage_rows)])

    return k(rmsg, src, dst)


def _sc_pair_deg(src, dst, labels, freq_pad, n_cls, n_pad):
    E = src.shape[0]
    n_obj = labels.shape[0]
    Fp = freq_pad.shape[1]
    per_w = E // _NW
    nch = per_w // _C
    rows_per_tile = n_pad // _TILES
    stage_rows = rows_per_tile // 5

    @functools.partial(
        pl.kernel,
        mesh=_sc_mesh(),
        compiler_params=pltpu.CompilerParams(needs_layout_passes=False,
                                             use_tc_tiling_on_sc=False),
        out_type=[
            jax.ShapeDtypeStruct((E, Fp), F32),
            jax.ShapeDtypeStruct((2 * n_pad, 16), F32),
        ],
        scratch_types=[
            pltpu.VMEM((_C,), I32),
            pltpu.VMEM((_C,), I32),
            pltpu.VMEM((_C,), I32),
            pltpu.VMEM((n_obj,), I32),
            pltpu.VMEM((_C, Fp), F32),
            pltpu.VMEM((_C, 16), F32),
            pltpu.VMEM((stage_rows, 16), F32),
            pltpu.VMEM_SHARED((n_pad, 16), F32),
        ],
    )
    def k(src_hbm, dst_hbm, lab_hbm, freq_hbm, bias_hbm, deg_hbm,
          sidx, didx, pidx, labv, frows, ones, stage, deg):
        cid = lax.axis_index("c")
        sid = lax.axis_index("s")
        wid = sid * 2 + cid
        base = wid * per_w

        pltpu.sync_copy(lab_hbm, labv)

        def orow(r, carry):
            ones[r, :] = jnp.ones((16,), F32)
            return carry

        lax.fori_loop(0, _C, orow, 0)

        def zrow(r, carry):
            stage[r, :] = jnp.zeros((16,), F32)
            return carry

        lax.fori_loop(0, stage_rows, zrow, 0)
        for kk in range(5):
            r0 = sid * rows_per_tile + kk * stage_rows
            pltpu.sync_copy(stage, deg.at[pl.ds(r0, stage_rows)])
        plsc.subcore_barrier()

        def chunk(c, carry):
            off = pl.multiple_of(base + c * _C, 8)
            pltpu.sync_copy(src_hbm.at[pl.ds(off, _C)], sidx)
            pltpu.sync_copy(dst_hbm.at[pl.ds(off, _C)], didx)
            pltpu.sync_copy(ones, deg.at[sidx], add=True)
            pltpu.sync_copy(ones, deg.at[didx], add=True)
            for g in range(_C // 16):
                sl = pl.ds(g * 16, 16)
                ls = plsc.load_gather(labv, [sidx[sl]])
                ld = plsc.load_gather(labv, [didx[sl]])
                pidx[sl] = ls * n_cls + ld
            pltpu.sync_copy(freq_hbm.at[pidx], frows)
            pltpu.sync_copy(frows, bias_hbm.at[pl.ds(off, _C)])
            return carry

        lax.fori_loop(0, nch, chunk, 0)
        plsc.subcore_barrier()

        for kk in range(5):
            r0 = sid * rows_per_tile + kk * stage_rows
            pltpu.sync_copy(deg.at[pl.ds(r0, stage_rows)], stage)
            pltpu.sync_copy(stage, deg_hbm.at[pl.ds(cid * n_pad + r0, stage_rows)])

    return k(src, dst, labels, freq_pad)


_OBJ_BLK = 1280
_REL_BLK = 2560


def _tc_obj0(roi, W_obj, W_s, W_o):
    n, d = roi.shape
    dh = W_obj.shape[1]
    grid = n // _OBJ_BLK

    def body(x_ref, wobj_ref, ws_ref, wo_ref, h_ref, s_ref, o_ref):
        h = jnp.maximum(
            jnp.dot(x_ref[...], wobj_ref[...], preferred_element_type=F32), 0.0)
        h_ref[...] = h
        s_ref[...] = jnp.dot(h, ws_ref[...], preferred_element_type=F32)
        o_ref[...] = jnp.dot(h, wo_ref[...], preferred_element_type=F32)

    w_spec = pl.BlockSpec((d, dh), lambda i: (0, 0))
    row_spec = pl.BlockSpec((_OBJ_BLK, dh), lambda i: (i, 0))
    return pl.pallas_call(
        body,
        grid=(grid,),
        in_specs=[pl.BlockSpec((_OBJ_BLK, d), lambda i: (i, 0)),
                  w_spec, w_spec, w_spec],
        out_specs=[row_spec, row_spec, row_spec],
        out_shape=[jax.ShapeDtypeStruct((n, dh), F32)] * 3,
    )(roi, W_obj, W_s, W_o)


def _tc_rel_first(union, msg, W_rel, W_r):
    e, d = union.shape
    dh = W_rel.shape[1]
    grid = e // _REL_BLK

    def body(u_ref, m_ref, wrel_ref, wr_ref, h_ref, r_ref):
        h0 = jnp.maximum(
            jnp.dot(u_ref[...], wrel_ref[...], preferred_element_type=F32), 0.0)
        h = jnp.maximum(h0 + m_ref[...], 0.0)
        h_ref[...] = h
        r_ref[...] = jnp.dot(h, wr_ref[...], preferred_element_type=F32)

    blk = pl.BlockSpec((_REL_BLK, dh), lambda i: (i, 0))
    w_spec = pl.BlockSpec((d, dh), lambda i: (0, 0))
    return pl.pallas_call(
        body,
        grid=(grid,),
        in_specs=[pl.BlockSpec((_REL_BLK, d), lambda i: (i, 0)), blk,
                  w_spec, pl.BlockSpec((dh, dh), lambda i: (0, 0))],
        out_specs=[blk, blk],
        out_shape=[jax.ShapeDtypeStruct((e, dh), F32)] * 2,
    )(union, msg, W_rel, W_r)


def _tc_rel_next(rel_h, msg, W_r):
    e, dh = rel_h.shape
    grid = e // _REL_BLK

    def body(rh_ref, m_ref, wr_ref, r_ref):
        h = jnp.maximum(rh_ref[...] + m_ref[...], 0.0)
        r_ref[...] = jnp.dot(h, wr_ref[...], preferred_element_type=F32)

    blk = pl.BlockSpec((_REL_BLK, dh), lambda i: (i, 0))
    return pl.pallas_call(
        body,
        grid=(grid,),
        in_specs=[blk, blk, pl.BlockSpec((dh, dh), lambda i: (0, 0))],
        out_specs=blk,
        out_shape=jax.ShapeDtypeStruct((e, dh), F32),
    )(rel_h, msg, W_r)


def _tc_obj_update(obj_h, aggp, degp, W_s, W_o):
    n, dh = obj_h.shape
    grid = n // _OBJ_BLK

    def body(h_ref, a_ref, d_ref, ws_ref, wo_ref, h2_ref, s_ref, o_ref):
        a = a_ref[0] + a_ref[1]
        deg = jnp.maximum(d_ref[0, :, 0:1] + d_ref[1, :, 0:1], 1.0)
        h = jnp.maximum(h_ref[...] + a / deg, 0.0)
        h2_ref[...] = h
        s_ref[...] = jnp.dot(h, ws_ref[...], preferred_element_type=F32)
        o_ref[...] = jnp.dot(h, wo_ref[...], preferred_element_type=F32)

    blk = pl.BlockSpec((_OBJ_BLK, dh), lambda i: (i, 0))
    w_spec = pl.BlockSpec((dh, dh), lambda i: (0, 0))
    return pl.pallas_call(
        body,
        grid=(grid,),
        in_specs=[blk,
                  pl.BlockSpec((2, _OBJ_BLK, dh), lambda i: (0, i, 0)),
                  pl.BlockSpec((2, _OBJ_BLK, 16), lambda i: (0, i, 0)),
                  w_spec, w_spec],
        out_specs=[blk, blk, blk],
        out_shape=[jax.ShapeDtypeStruct((n, dh), F32)] * 3,
    )(obj_h, aggp, degp, W_s, W_o)


def _tc_obj_final(obj_h, aggp, degp, W_cls_pad):
    n, dh = obj_h.shape
    ncp = W_cls_pad.shape[1]
    grid = n // _OBJ_BLK

    def body(h_ref, a_ref, d_ref, wc_ref, out_ref):
        a = a_ref[0] + a_ref[1]
        deg = jnp.maximum(d_ref[0, :, 0:1] + d_ref[1, :, 0:1], 1.0)
        h = jnp.maximum(h_ref[...] + a / deg, 0.0)
        out_ref[...] = jnp.dot(h, wc_ref[...], preferred_element_type=F32)

    blk = pl.BlockSpec((_OBJ_BLK, dh), lambda i: (i, 0))
    return pl.pallas_call(
        body,
        grid=(grid,),
        in_specs=[blk,
                  pl.BlockSpec((2, _OBJ_BLK, dh), lambda i: (0, i, 0)),
                  pl.BlockSpec((2, _OBJ_BLK, 16), lambda i: (0, i, 0)),
                  pl.BlockSpec((dh, ncp), lambda i: (0, 0))],
        out_specs=pl.BlockSpec((_OBJ_BLK, ncp), lambda i: (i, 0)),
        out_shape=jax.ShapeDtypeStruct((n, ncp), F32),
    )(obj_h, aggp, degp, W_cls_pad)


def _tc_rel_cls(rel_h, msg, bias, W_cls_pad):
    e, dh = rel_h.shape
    ncp = W_cls_pad.shape[1]
    grid = e // _REL_BLK

    def body(rh_ref, m_ref, b_ref, wc_ref, out_ref):
        h = jnp.maximum(rh_ref[...] + m_ref[...], 0.0)
        out_ref[...] = jnp.dot(h, wc_ref[...], preferred_element_type=F32) + b_ref[...]

    blk = pl.BlockSpec((_REL_BLK, dh), lambda i: (i, 0))
    bblk = pl.BlockSpec((_REL_BLK, ncp), lambda i: (i, 0))
    return pl.pallas_call(
        body,
        grid=(grid,),
        in_specs=[blk, blk, bblk, pl.BlockSpec((dh, ncp), lambda i: (0, 0))],
        out_specs=bblk,
        out_shape=jax.ShapeDtypeStruct((e, ncp), F32),
    )(rel_h, msg, bias, W_cls_pad)


def kernel(roi_features, union_features, rel_pair_idxs, obj_pred_labels,
           W_obj, W_rel, W_s, W_o, W_r, W_obj_cls, W_rel_cls, freq_table):
    n_obj, d_in = roi_features.shape
    E = union_features.shape[0]
    dh = W_obj.shape[1]
    n_obj_cls = W_obj_cls.shape[1]
    n_rel_cls = W_rel_cls.shape[1]
    n_cls = math.isqrt(freq_table.shape[0])

    n_pad = ((n_obj + _TILES * 40 - 1) // (_TILES * 40)) * (_TILES * 40)

    src = jnp.asarray(rel_pair_idxs[:, 0])
    dst = jnp.asarray(rel_pair_idxs[:, 1])
    roi_pad = jnp.pad(roi_features, ((0, n_pad - n_obj), (0, 0)))

    fp = 64
    freq_pad = jnp.pad(freq_table, ((0, 0), (0, fp - n_rel_cls)))
    ocp = ((n_obj_cls + 7) // 8) * 8
    W_obj_cls_pad = jnp.pad(W_obj_cls, ((0, 0), (0, ocp - n_obj_cls)))
    W_rel_cls_pad = jnp.pad(W_rel_cls, ((0, 0), (0, fp - n_rel_cls)))

    obj_h, S, O = _tc_obj0(roi_pad, W_obj, W_s, W_o)

    bias, degp_flat = _sc_pair_deg(src, dst, obj_pred_labels, freq_pad,
                                   n_cls, n_pad)
    degp = degp_flat.reshape(2, n_pad, 16)

    msg0 = _sc_gather_sum(S, O, src, dst)
    rel_h1, rmsg0 = _tc_rel_first(union_features, msg0, W_rel, W_r)
    aggp0 = _sc_scatter_both(rmsg0, src, dst, n_pad).reshape(2, n_pad, dh)
    obj_h1, S1, O1 = _tc_obj_update(obj_h, aggp0, degp, W_s, W_o)

    msg1 = _sc_gather_sum(S1, O1, src, dst)
    rmsg1 = _tc_rel_next(rel_h1, msg1, W_r)
    aggp1 = _sc_scatter_both(rmsg1, src, dst, n_pad).reshape(2, n_pad, dh)
    obj_logits = _tc_obj_final(obj_h1, aggp1, degp,
                               W_obj_cls_pad)[:n_obj, :n_obj_cls]

    rel_logits = _tc_rel_cls(rel_h1, msg1, bias, W_rel_cls_pad)[:, :n_rel_cls]

    return obj_logits, rel_logits


# speedup vs baseline: 6.8336x; 1.1867x over previous
import functools
import math

import jax
import jax.numpy as jnp
from jax import lax
from jax.experimental import pallas as pl
from jax.experimental.pallas import tpu as pltpu
from jax.experimental.pallas import tpu_sc as plsc

F32 = jnp.float32
I32 = jnp.int32

_NW = 32
_TILES = 16
_C = 80


def _sc_mesh():
    return plsc.VectorSubcoreMesh(core_axis_name="c", subcore_axis_name="s")


def _sc_gather_sum(S, O, src, dst):
    E = src.shape[0]
    D = S.shape[1]
    per_w = E // _NW
    nch = per_w // _C

    nb = 3

    @functools.partial(
        pl.kernel,
        mesh=_sc_mesh(),
        compiler_params=pltpu.CompilerParams(needs_layout_passes=False),
        out_type=jax.ShapeDtypeStruct((E, D), F32),
        scratch_types=(
            [pltpu.VMEM((_C,), I32) for _ in range(2 * nb)]
            + [pltpu.VMEM((_C, D), F32) for _ in range(2 * nb)]
            + [pltpu.SemaphoreType.DMA for _ in range(3 * nb)]
        ),
    )
    def k(S_hbm, O_hbm, src_hbm, dst_hbm, out_hbm, *scr):
        sidx = scr[0:nb]
        didx = scr[nb:2 * nb]
        ra = scr[2 * nb:3 * nb]
        rb = scr[3 * nb:4 * nb]
        si = scr[4 * nb:5 * nb]
        sg = scr[5 * nb:6 * nb]
        sw = scr[6 * nb:7 * nb]
        cid = lax.axis_index("c")
        sid = lax.axis_index("s")
        wid = sid * 2 + cid
        base = wid * per_w

        def off_of(kc):
            return pl.multiple_of(base + kc * _C, 8)

        def issue_idx(kc, b):
            pltpu.async_copy(src_hbm.at[pl.ds(off_of(kc), _C)], sidx[b], si[b])
            pltpu.async_copy(dst_hbm.at[pl.ds(off_of(kc), _C)], didx[b], si[b])

        def add_and_write(kc, b):
            def addrow(r, c2):
                for g in range(D // 16):
                    sl = pl.ds(g * 16, 16)
                    ra[b][r, sl] = ra[b][r, sl] + rb[b][r, sl]
                return c2

            lax.fori_loop(0, _C, addrow, 0)
            pltpu.async_copy(ra[b], out_hbm.at[pl.ds(off_of(kc), _C)], sw[b])

        issue_idx(0, 0)
        issue_idx(1, 1)

        n_outer = (nch + nb - 1) // nb

        def outer(j, carry):
            for b in range(nb):
                kc = j * nb + b
                bp = (b - 1) % nb

                @pl.when(kc < nch)
                def _():
                    pltpu.make_async_copy(
                        src_hbm.at[pl.ds(off_of(kc), _C)], sidx[b], si[b]).wait()
                    pltpu.make_async_copy(
                        dst_hbm.at[pl.ds(off_of(kc), _C)], didx[b], si[b]).wait()

                    @pl.when(kc >= nb)
                    def _():
                        pltpu.make_async_copy(
                            ra[b], out_hbm.at[pl.ds(off_of(kc), _C)],
                            sw[b]).wait()

                    pltpu.async_copy(S_hbm.at[sidx[b]], ra[b], sg[b])
                    pltpu.async_copy(O_hbm.at[didx[b]], rb[b], sg[b])

                    @pl.when(kc >= 1)
                    def _():
                        pltpu.make_async_copy(
                            S_hbm.at[sidx[bp]], ra[bp], sg[bp]).wait()
                        pltpu.make_async_copy(
                            O_hbm.at[didx[bp]], rb[bp], sg[bp]).wait()
                        add_and_write(kc - 1, bp)

                    @pl.when(kc + 2 < nch)
                    def _():
                        issue_idx(kc + 2, bp)
            return carry

        lax.fori_loop(0, n_outer, outer, 0)

        bl = (nch - 1) % nb
        pltpu.make_async_copy(S_hbm.at[sidx[bl]], ra[bl], sg[bl]).wait()
        pltpu.make_async_copy(O_hbm.at[didx[bl]], rb[bl], sg[bl]).wait()
        add_and_write(nch - 1, bl)
        for t in range(nb):
            bw = (nch - 1 - t) % nb
            pltpu.make_async_copy(
                ra[bw], out_hbm.at[pl.ds(off_of(nch - 1 - t), _C)],
                sw[bw]).wait()

    return k(S, O, src, dst)


def _sc_scatter_both(rmsg, src, dst, n_pad):
    E, D = rmsg.shape
    per_w = E // _NW
    nch = per_w // _C
    rows_per_tile = n_pad // _TILES
    nwb = rows_per_tile // _C

    nb = 4

    @functools.partial(
        pl.kernel,
        mesh=_sc_mesh(),
        compiler_params=pltpu.CompilerParams(needs_layout_passes=False),
        out_type=jax.ShapeDtypeStruct((2 * n_pad, D), F32),
        scratch_types=(
            [pltpu.VMEM((_C,), I32) for _ in range(2 * nb)]
            + [pltpu.VMEM((_C, D), F32) for _ in range(nb)]
            + [pltpu.VMEM_SHARED((n_pad, D), F32)]
            + [pltpu.SemaphoreType.DMA for _ in range(2 * nb)]
        ),
    )
    def k(rmsg_hbm, src_hbm, dst_hbm, out_hbm, *scr):
        sidx = scr[0:nb]
        didx = scr[nb:2 * nb]
        rows = scr[2 * nb:3 * nb]
        agg = scr[3 * nb]
        sl = scr[3 * nb + 1:3 * nb + 1 + nb]
        ss = scr[3 * nb + 1 + nb:3 * nb + 1 + 2 * nb]
        cid = lax.axis_index("c")
        sid = lax.axis_index("s")
        wid = sid * 2 + cid
        base = wid * per_w

        def off_of(kc):
            return pl.multiple_of(base + kc * _C, 8)

        def issue_loads(kc, b):
            pltpu.async_copy(src_hbm.at[pl.ds(off_of(kc), _C)], sidx[b], sl[b])
            pltpu.async_copy(dst_hbm.at[pl.ds(off_of(kc), _C)], didx[b], sl[b])
            pltpu.async_copy(rmsg_hbm.at[pl.ds(off_of(kc), _C)], rows[b], sl[b])

        def wait_loads(kc, b):
            pltpu.make_async_copy(
                src_hbm.at[pl.ds(off_of(kc), _C)], sidx[b], sl[b]).wait()
            pltpu.make_async_copy(
                dst_hbm.at[pl.ds(off_of(kc), _C)], didx[b], sl[b]).wait()
            pltpu.make_async_copy(
                rmsg_hbm.at[pl.ds(off_of(kc), _C)], rows[b], sl[b]).wait()

        def wait_scatters(b):
            pltpu.make_async_copy(rows[b], agg.at[sidx[b]], ss[b]).wait()
            pltpu.make_async_copy(rows[b], agg.at[didx[b]], ss[b]).wait()

        def zrow(r, carry):
            for g in range(D // 16):
                rows[0][r, pl.ds(g * 16, 16)] = jnp.zeros((16,), F32)
            return carry

        lax.fori_loop(0, _C, zrow, 0)
        for kk in range(nwb):
            r0 = sid * rows_per_tile + kk * _C
            pltpu.sync_copy(rows[0], agg.at[pl.ds(r0, _C)])
        plsc.subcore_barrier()

        issue_loads(0, 0)
        issue_loads(1, 1)

        n_outer = (nch + nb - 1) // nb

        def outer(j, carry):
            for b in range(nb):
                kc = j * nb + b
                bq = (b + 2) % nb

                @pl.when(kc < nch)
                def _():
                    wait_loads(kc, b)
                    pltpu.async_copy(rows[b], agg.at[sidx[b]], ss[b], add=True)
                    pltpu.async_copy(rows[b], agg.at[didx[b]], ss[b], add=True)

                    @pl.when(kc >= 2)
                    def _():
                        wait_scatters(bq)

                    @pl.when(kc + 2 < nch)
                    def _():
                        issue_loads(kc + 2, bq)
            return carry

        lax.fori_loop(0, n_outer, outer, 0)
        wait_scatters((nch - 2) % nb)
        wait_scatters((nch - 1) % nb)
        plsc.subcore_barrier()

        for kk in range(nwb):
            r0 = sid * rows_per_tile + kk * _C
            pltpu.sync_copy(agg.at[pl.ds(r0, _C)], rows[0])
            pltpu.sync_copy(rows[0], out_hbm.at[pl.ds(cid * n_pad + r0, _C)])

    return k(rmsg, src, dst)


def _sc_pair_deg(src, dst, labels, freq_pad, n_cls, n_pad):
    E = src.shape[0]
    n_obj = labels.shape[0]
    Fp = freq_pad.shape[1]
    per_w = E // _NW
    nch = per_w // _C
    rows_per_tile = n_pad // _TILES
    stage_rows = rows_per_tile // 5

    @functools.partial(
        pl.kernel,
        mesh=_sc_mesh(),
        compiler_params=pltpu.CompilerParams(needs_layout_passes=False,
                                             use_tc_tiling_on_sc=False),
        out_type=[
            jax.ShapeDtypeStruct((E, Fp), F32),
            jax.ShapeDtypeStruct((2 * n_pad, 16), F32),
        ],
        scratch_types=[
            pltpu.VMEM((_C,), I32),
            pltpu.VMEM((_C,), I32),
            pltpu.VMEM((_C,), I32),
            pltpu.VMEM((n_obj,), I32),
            pltpu.VMEM((_C, Fp), F32),
            pltpu.VMEM((_C, 16), F32),
            pltpu.VMEM((stage_rows, 16), F32),
            pltpu.VMEM_SHARED((n_pad, 16), F32),
        ],
    )
    def k(src_hbm, dst_hbm, lab_hbm, freq_hbm, bias_hbm, deg_hbm,
          sidx, didx, pidx, labv, frows, ones, stage, deg):
        cid = lax.axis_index("c")
        sid = lax.axis_index("s")
        wid = sid * 2 + cid
        base = wid * per_w

        pltpu.sync_copy(lab_hbm, labv)

        def orow(r, carry):
            ones[r, :] = jnp.ones((16,), F32)
            return carry

        lax.fori_loop(0, _C, orow, 0)

        def zrow(r, carry):
            stage[r, :] = jnp.zeros((16,), F32)
            return carry

        lax.fori_loop(0, stage_rows, zrow, 0)
        for kk in range(5):
            r0 = sid * rows_per_tile + kk * stage_rows
            pltpu.sync_copy(stage, deg.at[pl.ds(r0, stage_rows)])
        plsc.subcore_barrier()

        def chunk(c, carry):
            off = pl.multiple_of(base + c * _C, 8)
            pltpu.sync_copy(src_hbm.at[pl.ds(off, _C)], sidx)
            pltpu.sync_copy(dst_hbm.at[pl.ds(off, _C)], didx)
            pltpu.sync_copy(ones, deg.at[sidx], add=True)
            pltpu.sync_copy(ones, deg.at[didx], add=True)
            for g in range(_C // 16):
                sl = pl.ds(g * 16, 16)
                ls = plsc.load_gather(labv, [sidx[sl]])
                ld = plsc.load_gather(labv, [didx[sl]])
                pidx[sl] = ls * n_cls + ld
            pltpu.sync_copy(freq_hbm.at[pidx], frows)
            pltpu.sync_copy(frows, bias_hbm.at[pl.ds(off, _C)])
            return carry

        lax.fori_loop(0, nch, chunk, 0)
        plsc.subcore_barrier()

        for kk in range(5):
            r0 = sid * rows_per_tile + kk * stage_rows
            pltpu.sync_copy(deg.at[pl.ds(r0, stage_rows)], stage)
            pltpu.sync_copy(stage, deg_hbm.at[pl.ds(cid * n_pad + r0, stage_rows)])

    return k(src, dst, labels, freq_pad)


_OBJ_BLK = 1280
_REL_BLK = 2560


def _tc_obj0(roi, W_obj, W_s, W_o):
    n, d = roi.shape
    dh = W_obj.shape[1]
    grid = n // _OBJ_BLK

    def body(x_ref, wobj_ref, ws_ref, wo_ref, h_ref, s_ref, o_ref):
        h = jnp.maximum(
            jnp.dot(x_ref[...], wobj_ref[...], preferred_element_type=F32), 0.0)
        h_ref[...] = h
        s_ref[...] = jnp.dot(h, ws_ref[...], preferred_element_type=F32)
        o_ref[...] = jnp.dot(h, wo_ref[...], preferred_element_type=F32)

    w_spec = pl.BlockSpec((d, dh), lambda i: (0, 0))
    row_spec = pl.BlockSpec((_OBJ_BLK, dh), lambda i: (i, 0))
    return pl.pallas_call(
        body,
        grid=(grid,),
        in_specs=[pl.BlockSpec((_OBJ_BLK, d), lambda i: (i, 0)),
                  w_spec, w_spec, w_spec],
        out_specs=[row_spec, row_spec, row_spec],
        out_shape=[jax.ShapeDtypeStruct((n, dh), F32)] * 3,
    )(roi, W_obj, W_s, W_o)


def _tc_rel_first(union, msg, W_rel, W_r):
    e, d = union.shape
    dh = W_rel.shape[1]
    grid = e // _REL_BLK

    def body(u_ref, m_ref, wrel_ref, wr_ref, h_ref, r_ref):
        h0 = jnp.maximum(
            jnp.dot(u_ref[...], wrel_ref[...], preferred_element_type=F32), 0.0)
        h = jnp.maximum(h0 + m_ref[...], 0.0)
        h_ref[...] = h
        r_ref[...] = jnp.dot(h, wr_ref[...], preferred_element_type=F32)

    blk = pl.BlockSpec((_REL_BLK, dh), lambda i: (i, 0))
    w_spec = pl.BlockSpec((d, dh), lambda i: (0, 0))
    return pl.pallas_call(
        body,
        grid=(grid,),
        in_specs=[pl.BlockSpec((_REL_BLK, d), lambda i: (i, 0)), blk,
                  w_spec, pl.BlockSpec((dh, dh), lambda i: (0, 0))],
        out_specs=[blk, blk],
        out_shape=[jax.ShapeDtypeStruct((e, dh), F32)] * 2,
    )(union, msg, W_rel, W_r)


def _tc_rel_next(rel_h, msg, W_r):
    e, dh = rel_h.shape
    grid = e // _REL_BLK

    def body(rh_ref, m_ref, wr_ref, r_ref):
        h = jnp.maximum(rh_ref[...] + m_ref[...], 0.0)
        r_ref[...] = jnp.dot(h, wr_ref[...], preferred_element_type=F32)

    blk = pl.BlockSpec((_REL_BLK, dh), lambda i: (i, 0))
    return pl.pallas_call(
        body,
        grid=(grid,),
        in_specs=[blk, blk, pl.BlockSpec((dh, dh), lambda i: (0, 0))],
        out_specs=blk,
        out_shape=jax.ShapeDtypeStruct((e, dh), F32),
    )(rel_h, msg, W_r)


def _tc_obj_update(obj_h, aggp, degp, W_s, W_o):
    n, dh = obj_h.shape
    grid = n // _OBJ_BLK

    def body(h_ref, a_ref, d_ref, ws_ref, wo_ref, h2_ref, s_ref, o_ref):
        a = a_ref[0] + a_ref[1]
        deg = jnp.maximum(d_ref[0, :, 0:1] + d_ref[1, :, 0:1], 1.0)
        h = jnp.maximum(h_ref[...] + a / deg, 0.0)
        h2_ref[...] = h
        s_ref[...] = jnp.dot(h, ws_ref[...], preferred_element_type=F32)
        o_ref[...] = jnp.dot(h, wo_ref[...], preferred_element_type=F32)

    blk = pl.BlockSpec((_OBJ_BLK, dh), lambda i: (i, 0))
    w_spec = pl.BlockSpec((dh, dh), lambda i: (0, 0))
    return pl.pallas_call(
        body,
        grid=(grid,),
        in_specs=[blk,
                  pl.BlockSpec((2, _OBJ_BLK, dh), lambda i: (0, i, 0)),
                  pl.BlockSpec((2, _OBJ_BLK, 16), lambda i: (0, i, 0)),
                  w_spec, w_spec],
        out_specs=[blk, blk, blk],
        out_shape=[jax.ShapeDtypeStruct((n, dh), F32)] * 3,
    )(obj_h, aggp, degp, W_s, W_o)


def _tc_obj_final(obj_h, aggp, degp, W_cls_pad):
    n, dh = obj_h.shape
    ncp = W_cls_pad.shape[1]
    grid = n // _OBJ_BLK

    def body(h_ref, a_ref, d_ref, wc_ref, out_ref):
        a = a_ref[0] + a_ref[1]
        deg = jnp.maximum(d_ref[0, :, 0:1] + d_ref[1, :, 0:1], 1.0)
        h = jnp.maximum(h_ref[...] + a / deg, 0.0)
        out_ref[...] = jnp.dot(h, wc_ref[...], preferred_element_type=F32)

    blk = pl.BlockSpec((_OBJ_BLK, dh), lambda i: (i, 0))
    return pl.pallas_call(
        body,
        grid=(grid,),
        in_specs=[blk,
                  pl.BlockSpec((2, _OBJ_BLK, dh), lambda i: (0, i, 0)),
                  pl.BlockSpec((2, _OBJ_BLK, 16), lambda i: (0, i, 0)),
                  pl.BlockSpec((dh, ncp), lambda i: (0, 0))],
        out_specs=pl.BlockSpec((_OBJ_BLK, ncp), lambda i: (i, 0)),
        out_shape=jax.ShapeDtypeStruct((n, ncp), F32),
    )(obj_h, aggp, degp, W_cls_pad)


def _tc_rel_cls(rel_h, msg, bias, W_cls_pad):
    e, dh = rel_h.shape
    ncp = W_cls_pad.shape[1]
    grid = e // _REL_BLK

    def body(rh_ref, m_ref, b_ref, wc_ref, out_ref):
        h = jnp.maximum(rh_ref[...] + m_ref[...], 0.0)
        out_ref[...] = jnp.dot(h, wc_ref[...], preferred_element_type=F32) + b_ref[...]

    blk = pl.BlockSpec((_REL_BLK, dh), lambda i: (i, 0))
    bblk = pl.BlockSpec((_REL_BLK, ncp), lambda i: (i, 0))
    return pl.pallas_call(
        body,
        grid=(grid,),
        in_specs=[blk, blk, bblk, pl.BlockSpec((dh, ncp), lambda i: (0, 0))],
        out_specs=bblk,
        out_shape=jax.ShapeDtypeStruct((e, ncp), F32),
    )(rel_h, msg, bias, W_cls_pad)


def kernel(roi_features, union_features, rel_pair_idxs, obj_pred_labels,
           W_obj, W_rel, W_s, W_o, W_r, W_obj_cls, W_rel_cls, freq_table):
    n_obj, d_in = roi_features.shape
    E = union_features.shape[0]
    dh = W_obj.shape[1]
    n_obj_cls = W_obj_cls.shape[1]
    n_rel_cls = W_rel_cls.shape[1]
    n_cls = math.isqrt(freq_table.shape[0])

    n_pad = ((n_obj + _TILES * 40 - 1) // (_TILES * 40)) * (_TILES * 40)

    src = jnp.asarray(rel_pair_idxs[:, 0])
    dst = jnp.asarray(rel_pair_idxs[:, 1])
    roi_pad = jnp.pad(roi_features, ((0, n_pad - n_obj), (0, 0)))

    fp = 64
    freq_pad = jnp.pad(freq_table, ((0, 0), (0, fp - n_rel_cls)))
    ocp = ((n_obj_cls + 7) // 8) * 8
    W_obj_cls_pad = jnp.pad(W_obj_cls, ((0, 0), (0, ocp - n_obj_cls)))
    W_rel_cls_pad = jnp.pad(W_rel_cls, ((0, 0), (0, fp - n_rel_cls)))

    obj_h, S, O = _tc_obj0(roi_pad, W_obj, W_s, W_o)

    bias, degp_flat = _sc_pair_deg(src, dst, obj_pred_labels, freq_pad,
                                   n_cls, n_pad)
    degp = degp_flat.reshape(2, n_pad, 16)

    msg0 = _sc_gather_sum(S, O, src, dst)
    rel_h1, rmsg0 = _tc_rel_first(union_features, msg0, W_rel, W_r)
    aggp0 = _sc_scatter_both(rmsg0, src, dst, n_pad).reshape(2, n_pad, dh)
    obj_h1, S1, O1 = _tc_obj_update(obj_h, aggp0, degp, W_s, W_o)

    msg1 = _sc_gather_sum(S1, O1, src, dst)
    rmsg1 = _tc_rel_next(rel_h1, msg1, W_r)
    aggp1 = _sc_scatter_both(rmsg1, src, dst, n_pad).reshape(2, n_pad, dh)
    obj_logits = _tc_obj_final(obj_h1, aggp1, degp,
                               W_obj_cls_pad)[:n_obj, :n_obj_cls]

    rel_logits = _tc_rel_cls(rel_h1, msg1, bias, W_rel_cls_pad)[:, :n_rel_cls]

    return obj_logits, rel_logits


# speedup vs baseline: 7.3881x; 1.0812x over previous
import functools
import math

import jax
import jax.numpy as jnp
from jax import lax
from jax.experimental import pallas as pl
from jax.experimental.pallas import tpu as pltpu
from jax.experimental.pallas import tpu_sc as plsc

F32 = jnp.float32
I32 = jnp.int32

_NW = 32
_TILES = 16
_C = 80


def _sc_mesh():
    return plsc.VectorSubcoreMesh(core_axis_name="c", subcore_axis_name="s")


def _sc_gather_sum(S, O, src, dst):
    E = src.shape[0]
    D = S.shape[1]
    per_w = E // _NW
    nch = per_w // _C

    nb = 3

    @functools.partial(
        pl.kernel,
        mesh=_sc_mesh(),
        compiler_params=pltpu.CompilerParams(needs_layout_passes=False),
        out_type=jax.ShapeDtypeStruct((E, D), F32),
        scratch_types=(
            [pltpu.VMEM((_C,), I32) for _ in range(2 * nb)]
            + [pltpu.VMEM((_C, D), F32) for _ in range(2 * nb)]
            + [pltpu.SemaphoreType.DMA for _ in range(3 * nb)]
        ),
    )
    def k(S_hbm, O_hbm, src_hbm, dst_hbm, out_hbm, *scr):
        sidx = scr[0:nb]
        didx = scr[nb:2 * nb]
        ra = scr[2 * nb:3 * nb]
        rb = scr[3 * nb:4 * nb]
        si = scr[4 * nb:5 * nb]
        sg = scr[5 * nb:6 * nb]
        sw = scr[6 * nb:7 * nb]
        cid = lax.axis_index("c")
        sid = lax.axis_index("s")
        wid = sid * 2 + cid
        base = wid * per_w

        def off_of(kc):
            return pl.multiple_of(base + kc * _C, 8)

        def issue_idx(kc, b):
            pltpu.async_copy(src_hbm.at[pl.ds(off_of(kc), _C)], sidx[b], si[b])
            pltpu.async_copy(dst_hbm.at[pl.ds(off_of(kc), _C)], didx[b], si[b])

        def add_and_write(kc, b):
            def addrow(r, c2):
                for g in range(D // 16):
                    sl = pl.ds(g * 16, 16)
                    ra[b][r, sl] = ra[b][r, sl] + rb[b][r, sl]
                return c2

            lax.fori_loop(0, _C, addrow, 0)
            pltpu.async_copy(ra[b], out_hbm.at[pl.ds(off_of(kc), _C)], sw[b])

        issue_idx(0, 0)
        issue_idx(1, 1)

        n_outer = (nch + nb - 1) // nb

        def outer(j, carry):
            for b in range(nb):
                kc = j * nb + b
                bp = (b - 1) % nb

                @pl.when(kc < nch)
                def _():
                    pltpu.make_async_copy(
                        src_hbm.at[pl.ds(off_of(kc), _C)], sidx[b], si[b]).wait()
                    pltpu.make_async_copy(
                        dst_hbm.at[pl.ds(off_of(kc), _C)], didx[b], si[b]).wait()

                    @pl.when(kc >= nb)
                    def _():
                        pltpu.make_async_copy(
                            ra[b], out_hbm.at[pl.ds(off_of(kc), _C)],
                            sw[b]).wait()

                    pltpu.async_copy(S_hbm.at[sidx[b]], ra[b], sg[b])
                    pltpu.async_copy(O_hbm.at[didx[b]], rb[b], sg[b])

                    @pl.when(kc >= 1)
                    def _():
                        pltpu.make_async_copy(
                            S_hbm.at[sidx[bp]], ra[bp], sg[bp]).wait()
                        pltpu.make_async_copy(
                            O_hbm.at[didx[bp]], rb[bp], sg[bp]).wait()
                        add_and_write(kc - 1, bp)

                    @pl.when(kc + 2 < nch)
                    def _():
                        issue_idx(kc + 2, bp)
            return carry

        lax.fori_loop(0, n_outer, outer, 0)

        bl = (nch - 1) % nb
        pltpu.make_async_copy(S_hbm.at[sidx[bl]], ra[bl], sg[bl]).wait()
        pltpu.make_async_copy(O_hbm.at[didx[bl]], rb[bl], sg[bl]).wait()
        add_and_write(nch - 1, bl)
        for t in range(nb):
            bw = (nch - 1 - t) % nb
            pltpu.make_async_copy(
                ra[bw], out_hbm.at[pl.ds(off_of(nch - 1 - t), _C)],
                sw[bw]).wait()

    return k(S, O, src, dst)


def _sc_scatter_both(rmsg, src, dst, n_pad):
    E, D = rmsg.shape
    per_w = E // _NW
    nch = per_w // _C
    rows_per_tile = n_pad // _TILES
    nwb = rows_per_tile // _C

    nb = 4

    @functools.partial(
        pl.kernel,
        mesh=_sc_mesh(),
        compiler_params=pltpu.CompilerParams(needs_layout_passes=False),
        out_type=jax.ShapeDtypeStruct((2 * n_pad, D), F32),
        scratch_types=(
            [pltpu.VMEM((_C,), I32) for _ in range(2 * nb)]
            + [pltpu.VMEM((_C, D), F32) for _ in range(nb)]
            + [pltpu.VMEM_SHARED((n_pad, D), F32)]
            + [pltpu.SemaphoreType.DMA for _ in range(2 * nb)]
        ),
    )
    def k(rmsg_hbm, src_hbm, dst_hbm, out_hbm, *scr):
        sidx = scr[0:nb]
        didx = scr[nb:2 * nb]
        rows = scr[2 * nb:3 * nb]
        agg = scr[3 * nb]
        sl = scr[3 * nb + 1:3 * nb + 1 + nb]
        ss = scr[3 * nb + 1 + nb:3 * nb + 1 + 2 * nb]
        cid = lax.axis_index("c")
        sid = lax.axis_index("s")
        wid = sid * 2 + cid
        base = wid * per_w

        def off_of(kc):
            return pl.multiple_of(base + kc * _C, 8)

        def issue_loads(kc, b):
            pltpu.async_copy(src_hbm.at[pl.ds(off_of(kc), _C)], sidx[b], sl[b])
            pltpu.async_copy(dst_hbm.at[pl.ds(off_of(kc), _C)], didx[b], sl[b])
            pltpu.async_copy(rmsg_hbm.at[pl.ds(off_of(kc), _C)], rows[b], sl[b])

        def wait_loads(kc, b):
            pltpu.make_async_copy(
                src_hbm.at[pl.ds(off_of(kc), _C)], sidx[b], sl[b]).wait()
            pltpu.make_async_copy(
                dst_hbm.at[pl.ds(off_of(kc), _C)], didx[b], sl[b]).wait()
            pltpu.make_async_copy(
                rmsg_hbm.at[pl.ds(off_of(kc), _C)], rows[b], sl[b]).wait()

        def wait_scatters(b):
            pltpu.make_async_copy(rows[b], agg.at[sidx[b]], ss[b]).wait()
            pltpu.make_async_copy(rows[b], agg.at[didx[b]], ss[b]).wait()

        def zrow(r, carry):
            for g in range(D // 16):
                rows[0][r, pl.ds(g * 16, 16)] = jnp.zeros((16,), F32)
            return carry

        lax.fori_loop(0, _C, zrow, 0)
        for kk in range(nwb):
            r0 = sid * rows_per_tile + kk * _C
            pltpu.sync_copy(rows[0], agg.at[pl.ds(r0, _C)])
        plsc.subcore_barrier()

        issue_loads(0, 0)
        issue_loads(1, 1)

        n_outer = (nch + nb - 1) // nb

        def outer(j, carry):
            for b in range(nb):
                kc = j * nb + b
                bq = (b + 2) % nb

                @pl.when(kc < nch)
                def _():
                    wait_loads(kc, b)
                    pltpu.async_copy(rows[b], agg.at[sidx[b]], ss[b], add=True)
                    pltpu.async_copy(rows[b], agg.at[didx[b]], ss[b], add=True)

                    @pl.when(kc >= 2)
                    def _():
                        wait_scatters(bq)

                    @pl.when(kc + 2 < nch)
                    def _():
                        issue_loads(kc + 2, bq)
            return carry

        lax.fori_loop(0, n_outer, outer, 0)
        wait_scatters((nch - 2) % nb)
        wait_scatters((nch - 1) % nb)
        plsc.subcore_barrier()

        for kk in range(nwb):
            r0 = sid * rows_per_tile + kk * _C
            pltpu.sync_copy(agg.at[pl.ds(r0, _C)], rows[0])
            pltpu.sync_copy(rows[0], out_hbm.at[pl.ds(cid * n_pad + r0, _C)])

    return k(rmsg, src, dst)


def _sc_pair_deg(src, dst, labels, freq_pad, n_cls, n_pad):
    E = src.shape[0]
    n_obj = labels.shape[0]
    Fp = freq_pad.shape[1]
    per_w = E // _NW
    nch = per_w // _C
    rows_per_tile = n_pad // _TILES
    stage_rows = rows_per_tile // 5

    nb = 3

    @functools.partial(
        pl.kernel,
        mesh=_sc_mesh(),
        compiler_params=pltpu.CompilerParams(needs_layout_passes=False,
                                             use_tc_tiling_on_sc=False),
        out_type=[
            jax.ShapeDtypeStruct((E, Fp), F32),
            jax.ShapeDtypeStruct((2 * n_pad, 16), F32),
        ],
        scratch_types=(
            [pltpu.VMEM((_C,), I32) for _ in range(3 * nb)]
            + [pltpu.VMEM((_C, Fp), F32) for _ in range(nb)]
            + [pltpu.VMEM((n_obj,), I32),
               pltpu.VMEM((_C, 16), F32),
               pltpu.VMEM((stage_rows, 16), F32),
               pltpu.VMEM_SHARED((n_pad, 16), F32)]
            + [pltpu.SemaphoreType.DMA for _ in range(4 * nb)]
        ),
    )
    def k(src_hbm, dst_hbm, lab_hbm, freq_hbm, bias_hbm, deg_hbm, *scr):
        sidx = scr[0:nb]
        didx = scr[nb:2 * nb]
        pidx = scr[2 * nb:3 * nb]
        frows = scr[3 * nb:4 * nb]
        labv = scr[4 * nb]
        ones = scr[4 * nb + 1]
        stage = scr[4 * nb + 2]
        deg = scr[4 * nb + 3]
        slp = scr[4 * nb + 4:4 * nb + 4 + nb]
        ssd = scr[4 * nb + 4 + nb:4 * nb + 4 + 2 * nb]
        sf = scr[4 * nb + 4 + 2 * nb:4 * nb + 4 + 3 * nb]
        sb = scr[4 * nb + 4 + 3 * nb:4 * nb + 4 + 4 * nb]
        cid = lax.axis_index("c")
        sid = lax.axis_index("s")
        wid = sid * 2 + cid
        base = wid * per_w

        pltpu.sync_copy(lab_hbm, labv)

        def orow(r, carry):
            ones[r, :] = jnp.ones((16,), F32)
            return carry

        lax.fori_loop(0, _C, orow, 0)

        def zrow(r, carry):
            stage[r, :] = jnp.zeros((16,), F32)
            return carry

        lax.fori_loop(0, stage_rows, zrow, 0)
        for kk in range(5):
            r0 = sid * rows_per_tile + kk * stage_rows
            pltpu.sync_copy(stage, deg.at[pl.ds(r0, stage_rows)])
        plsc.subcore_barrier()

        def off_of(kc):
            return pl.multiple_of(base + kc * _C, 8)

        def issue_loads(kc, b):
            pltpu.async_copy(src_hbm.at[pl.ds(off_of(kc), _C)], sidx[b], slp[b])
            pltpu.async_copy(dst_hbm.at[pl.ds(off_of(kc), _C)], didx[b], slp[b])

        def wait_loads(kc, b):
            pltpu.make_async_copy(
                src_hbm.at[pl.ds(off_of(kc), _C)], sidx[b], slp[b]).wait()
            pltpu.make_async_copy(
                dst_hbm.at[pl.ds(off_of(kc), _C)], didx[b], slp[b]).wait()

        issue_loads(0, 0)
        issue_loads(1, 1)

        n_outer = (nch + nb - 1) // nb

        def outer(j, carry):
            for b in range(nb):
                kc = j * nb + b
                bp = (b - 1) % nb

                @pl.when(kc < nch)
                def _():
                    wait_loads(kc, b)
                    pltpu.async_copy(ones, deg.at[sidx[b]], ssd[b], add=True)
                    pltpu.async_copy(ones, deg.at[didx[b]], ssd[b], add=True)
                    for g in range(_C // 16):
                        sl = pl.ds(g * 16, 16)
                        ls = plsc.load_gather(labv, [sidx[b][sl]])
                        ld = plsc.load_gather(labv, [didx[b][sl]])
                        pidx[b][sl] = ls * n_cls + ld

                    @pl.when(kc >= nb)
                    def _():
                        pltpu.make_async_copy(
                            frows[b], bias_hbm.at[pl.ds(off_of(kc), _C)],
                            sb[b]).wait()

                    pltpu.async_copy(freq_hbm.at[pidx[b]], frows[b], sf[b])

                    @pl.when(kc >= 1)
                    def _():
                        pltpu.make_async_copy(
                            freq_hbm.at[pidx[bp]], frows[bp], sf[bp]).wait()
                        pltpu.async_copy(
                            frows[bp], bias_hbm.at[pl.ds(off_of(kc - 1), _C)],
                            sb[bp])
                        pltpu.make_async_copy(
                            ones, deg.at[sidx[bp]], ssd[bp]).wait()
                        pltpu.make_async_copy(
                            ones, deg.at[didx[bp]], ssd[bp]).wait()

                    @pl.when(kc + 2 < nch)
                    def _():
                        issue_loads(kc + 2, bp)
            return carry

        lax.fori_loop(0, n_outer, outer, 0)

        bl = (nch - 1) % nb
        pltpu.make_async_copy(freq_hbm.at[pidx[bl]], frows[bl], sf[bl]).wait()
        pltpu.async_copy(frows[bl], bias_hbm.at[pl.ds(off_of(nch - 1), _C)],
                         sb[bl])
        pltpu.make_async_copy(ones, deg.at[sidx[bl]], ssd[bl]).wait()
        pltpu.make_async_copy(ones, deg.at[didx[bl]], ssd[bl]).wait()
        for t in range(nb):
            bw = (nch - 1 - t) % nb
            pltpu.make_async_copy(
                frows[bw], bias_hbm.at[pl.ds(off_of(nch - 1 - t), _C)],
                sb[bw]).wait()
        plsc.subcore_barrier()

        for kk in range(5):
            r0 = sid * rows_per_tile + kk * stage_rows
            pltpu.sync_copy(deg.at[pl.ds(r0, stage_rows)], stage)
            pltpu.sync_copy(stage, deg_hbm.at[pl.ds(cid * n_pad + r0, stage_rows)])

    return k(src, dst, labels, freq_pad)


_OBJ_BLK = 1280
_REL_BLK = 2560


def _tc_obj0(roi, W_obj, W_s, W_o):
    n, d = roi.shape
    dh = W_obj.shape[1]
    grid = n // _OBJ_BLK

    def body(x_ref, wobj_ref, ws_ref, wo_ref, h_ref, s_ref, o_ref):
        h = jnp.maximum(
            jnp.dot(x_ref[...], wobj_ref[...], preferred_element_type=F32), 0.0)
        h_ref[...] = h
        s_ref[...] = jnp.dot(h, ws_ref[...], preferred_element_type=F32)
        o_ref[...] = jnp.dot(h, wo_ref[...], preferred_element_type=F32)

    w_spec = pl.BlockSpec((d, dh), lambda i: (0, 0))
    row_spec = pl.BlockSpec((_OBJ_BLK, dh), lambda i: (i, 0))
    return pl.pallas_call(
        body,
        grid=(grid,),
        in_specs=[pl.BlockSpec((_OBJ_BLK, d), lambda i: (i, 0)),
                  w_spec, w_spec, w_spec],
        out_specs=[row_spec, row_spec, row_spec],
        out_shape=[jax.ShapeDtypeStruct((n, dh), F32)] * 3,
    )(roi, W_obj, W_s, W_o)


def _tc_rel_first(union, msg, W_rel, W_r):
    e, d = union.shape
    dh = W_rel.shape[1]
    grid = e // _REL_BLK

    def body(u_ref, m_ref, wrel_ref, wr_ref, h_ref, r_ref):
        h0 = jnp.maximum(
            jnp.dot(u_ref[...], wrel_ref[...], preferred_element_type=F32), 0.0)
        h = jnp.maximum(h0 + m_ref[...], 0.0)
        h_ref[...] = h
        r_ref[...] = jnp.dot(h, wr_ref[...], preferred_element_type=F32)

    blk = pl.BlockSpec((_REL_BLK, dh), lambda i: (i, 0))
    w_spec = pl.BlockSpec((d, dh), lambda i: (0, 0))
    return pl.pallas_call(
        body,
        grid=(grid,),
        in_specs=[pl.BlockSpec((_REL_BLK, d), lambda i: (i, 0)), blk,
                  w_spec, pl.BlockSpec((dh, dh), lambda i: (0, 0))],
        out_specs=[blk, blk],
        out_shape=[jax.ShapeDtypeStruct((e, dh), F32)] * 2,
    )(union, msg, W_rel, W_r)


def _tc_rel_next(rel_h, msg, W_r):
    e, dh = rel_h.shape
    grid = e // _REL_BLK

    def body(rh_ref, m_ref, wr_ref, r_ref):
        h = jnp.maximum(rh_ref[...] + m_ref[...], 0.0)
        r_ref[...] = jnp.dot(h, wr_ref[...], preferred_element_type=F32)

    blk = pl.BlockSpec((_REL_BLK, dh), lambda i: (i, 0))
    return pl.pallas_call(
        body,
        grid=(grid,),
        in_specs=[blk, blk, pl.BlockSpec((dh, dh), lambda i: (0, 0))],
        out_specs=blk,
        out_shape=jax.ShapeDtypeStruct((e, dh), F32),
    )(rel_h, msg, W_r)


def _tc_obj_update(obj_h, aggp, degp, W_s, W_o):
    n, dh = obj_h.shape
    grid = n // _OBJ_BLK

    def body(h_ref, a_ref, d_ref, ws_ref, wo_ref, h2_ref, s_ref, o_ref):
        a = a_ref[0] + a_ref[1]
        deg = jnp.maximum(d_ref[0, :, 0:1] + d_ref[1, :, 0:1], 1.0)
        h = jnp.maximum(h_ref[...] + a / deg, 0.0)
        h2_ref[...] = h
        s_ref[...] = jnp.dot(h, ws_ref[...], preferred_element_type=F32)
        o_ref[...] = jnp.dot(h, wo_ref[...], preferred_element_type=F32)

    blk = pl.BlockSpec((_OBJ_BLK, dh), lambda i: (i, 0))
    w_spec = pl.BlockSpec((dh, dh), lambda i: (0, 0))
    return pl.pallas_call(
        body,
        grid=(grid,),
        in_specs=[blk,
                  pl.BlockSpec((2, _OBJ_BLK, dh), lambda i: (0, i, 0)),
                  pl.BlockSpec((2, _OBJ_BLK, 16), lambda i: (0, i, 0)),
                  w_spec, w_spec],
        out_specs=[blk, blk, blk],
        out_shape=[jax.ShapeDtypeStruct((n, dh), F32)] * 3,
    )(obj_h, aggp, degp, W_s, W_o)


def _tc_obj_final(obj_h, aggp, degp, W_cls_pad):
    n, dh = obj_h.shape
    ncp = W_cls_pad.shape[1]
    grid = n // _OBJ_BLK

    def body(h_ref, a_ref, d_ref, wc_ref, out_ref):
        a = a_ref[0] + a_ref[1]
        deg = jnp.maximum(d_ref[0, :, 0:1] + d_ref[1, :, 0:1], 1.0)
        h = jnp.maximum(h_ref[...] + a / deg, 0.0)
        out_ref[...] = jnp.dot(h, wc_ref[...], preferred_element_type=F32)

    blk = pl.BlockSpec((_OBJ_BLK, dh), lambda i: (i, 0))
    return pl.pallas_call(
        body,
        grid=(grid,),
        in_specs=[blk,
                  pl.BlockSpec((2, _OBJ_BLK, dh), lambda i: (0, i, 0)),
                  pl.BlockSpec((2, _OBJ_BLK, 16), lambda i: (0, i, 0)),
                  pl.BlockSpec((dh, ncp), lambda i: (0, 0))],
        out_specs=pl.BlockSpec((_OBJ_BLK, ncp), lambda i: (i, 0)),
        out_shape=jax.ShapeDtypeStruct((n, ncp), F32),
    )(obj_h, aggp, degp, W_cls_pad)


def _tc_rel_cls(rel_h, msg, bias, W_cls_pad):
    e, dh = rel_h.shape
    ncp = W_cls_pad.shape[1]
    grid = e // _REL_BLK

    def body(rh_ref, m_ref, b_ref, wc_ref, out_ref):
        h = jnp.maximum(rh_ref[...] + m_ref[...], 0.0)
        out_ref[...] = jnp.dot(h, wc_ref[...], preferred_element_type=F32) + b_ref[...]

    blk = pl.BlockSpec((_REL_BLK, dh), lambda i: (i, 0))
    bblk = pl.BlockSpec((_REL_BLK, ncp), lambda i: (i, 0))
    return pl.pallas_call(
        body,
        grid=(grid,),
        in_specs=[blk, blk, bblk, pl.BlockSpec((dh, ncp), lambda i: (0, 0))],
        out_specs=bblk,
        out_shape=jax.ShapeDtypeStruct((e, ncp), F32),
    )(rel_h, msg, bias, W_cls_pad)


def kernel(roi_features, union_features, rel_pair_idxs, obj_pred_labels,
           W_obj, W_rel, W_s, W_o, W_r, W_obj_cls, W_rel_cls, freq_table):
    n_obj, d_in = roi_features.shape
    E = union_features.shape[0]
    dh = W_obj.shape[1]
    n_obj_cls = W_obj_cls.shape[1]
    n_rel_cls = W_rel_cls.shape[1]
    n_cls = math.isqrt(freq_table.shape[0])

    n_pad = ((n_obj + _TILES * 40 - 1) // (_TILES * 40)) * (_TILES * 40)

    src = jnp.asarray(rel_pair_idxs[:, 0])
    dst = jnp.asarray(rel_pair_idxs[:, 1])
    roi_pad = jnp.pad(roi_features, ((0, n_pad - n_obj), (0, 0)))

    fp = 64
    freq_pad = jnp.pad(freq_table, ((0, 0), (0, fp - n_rel_cls)))
    ocp = ((n_obj_cls + 7) // 8) * 8
    W_obj_cls_pad = jnp.pad(W_obj_cls, ((0, 0), (0, ocp - n_obj_cls)))
    W_rel_cls_pad = jnp.pad(W_rel_cls, ((0, 0), (0, fp - n_rel_cls)))

    obj_h, S, O = _tc_obj0(roi_pad, W_obj, W_s, W_o)

    bias, degp_flat = _sc_pair_deg(src, dst, obj_pred_labels, freq_pad,
                                   n_cls, n_pad)
    degp = degp_flat.reshape(2, n_pad, 16)

    msg0 = _sc_gather_sum(S, O, src, dst)
    rel_h1, rmsg0 = _tc_rel_first(union_features, msg0, W_rel, W_r)
    aggp0 = _sc_scatter_both(rmsg0, src, dst, n_pad).reshape(2, n_pad, dh)
    obj_h1, S1, O1 = _tc_obj_update(obj_h, aggp0, degp, W_s, W_o)

    msg1 = _sc_gather_sum(S1, O1, src, dst)
    rmsg1 = _tc_rel_next(rel_h1, msg1, W_r)
    aggp1 = _sc_scatter_both(rmsg1, src, dst, n_pad).reshape(2, n_pad, dh)
    obj_logits = _tc_obj_final(obj_h1, aggp1, degp,
                               W_obj_cls_pad)[:n_obj, :n_obj_cls]

    rel_logits = _tc_rel_cls(rel_h1, msg1, bias, W_rel_cls_pad)[:, :n_rel_cls]

    return obj_logits, rel_logits
